# Initial kernel scaffold; baseline (speedup 1.0000x reference)
import jax
import jax.numpy as jnp
from jax.experimental import pallas as pl


def kernel(x, pe):
    raise NotImplementedError("write your pallas kernel here")


# speedup vs baseline: 6.8400x; 6.8400x over previous
import functools

import jax
import jax.numpy as jnp
from jax import lax
from jax.experimental import pallas as pl
from jax.experimental.pallas import tpu as pltpu
from jax.experimental.pallas import tpu_sc as plsc

D_MODEL = 128
MAX_POS = 255
SEQ = 512
PAD = 1024

NUM_CORES = 2
NUM_SUBCORES = 16
NUM_WORKERS = NUM_CORES * NUM_SUBCORES


def kernel(x, pe):
    B, S = x.shape
    assert S == SEQ and pe.shape == (SEQ, D_MODEL)
    rows_per_worker = S // NUM_WORKERS

    mesh = plsc.VectorSubcoreMesh(core_axis_name="c", subcore_axis_name="s")

    @functools.partial(
        pl.kernel,
        out_type=jax.ShapeDtypeStruct((B, S, S, D_MODEL), jnp.float32),
        mesh=mesh,
        scratch_types=[
            pltpu.VMEM_SHARED((PAD, D_MODEL), jnp.float32),
            pltpu.VMEM((D_MODEL,), jnp.float32),
            pltpu.VMEM((NUM_SUBCORES, D_MODEL), jnp.float32),
        ],
    )
    def sc_kernel(pe_hbm, out_hbm, p_sh, row_v, rep_v):
        c = lax.axis_index("c")
        s = lax.axis_index("s")

        pltpu.sync_copy(
            pe_hbm.at[pl.ds(s * 32, 32)], p_sh.at[pl.ds(256 + s * 32, 32)]
        )
        @pl.when(s == NUM_SUBCORES - 1)
        def _fix_last():
            pltpu.sync_copy(pe_hbm.at[2 * MAX_POS], p_sh.at[PAD - 257])

        def replicate(src_row):
            pltpu.sync_copy(pe_hbm.at[src_row], row_v)
            for c16 in range(D_MODEL // 16):
                v = row_v[pl.ds(c16 * 16, 16)]
                for r in range(NUM_SUBCORES):
                    rep_v[r, pl.ds(c16 * 16, 16)] = v

        replicate(0)
        pltpu.sync_copy(rep_v, p_sh.at[pl.ds(s * 16, 16)])
        replicate(2 * MAX_POS)
        pltpu.sync_copy(rep_v, p_sh.at[pl.ds(768 + s * 16, 16)])

        plsc.subcore_barrier()

        wid = s * NUM_CORES + c
        i_base = wid * rows_per_worker
        for b in range(B):
            for k in range(rows_per_worker):
                i = i_base + k
                pltpu.sync_copy(
                    p_sh.at[pl.ds((S - 1) - i, S)], out_hbm.at[b, i]
                )

    return sc_kernel(pe)
